# baseline (speedup 1.0000x reference)
import functools

import jax
import jax.numpy as jnp
from jax import lax
from jax.experimental import pallas as pl
from jax.experimental.pallas import tpu as pltpu
from jax.experimental.pallas import tpu_sc as plsc

_GRID = 128
_MID = 128.0

_C = 400
_G = 80
_SEG_FRACS = (5, 4)
_NGR = _C // _G
_GPR = _G // 16
_NC = 2
_NS = 16
_NW = _NC * _NS


def _body(ox_hbm, oy_hbm, oz_hbm, vx_hbm, vy_hbm, vz_hbm, cache_hbm, out_hbm,
          i00, i01, i02, i03, i04, i05,
          i10, i11, i12, i13, i14, i15,
          idx0_v, idx1_v, vals0_v, vals1_v, out0_v, out1_v,
          sem_in0, sem_in1, sem_g0, sem_g1, sem_out0, sem_out1):
    in_bufs = ((i00, i01, i02, i03, i04, i05),
               (i10, i11, i12, i13, i14, i15))
    idx_bufs = (idx0_v, idx1_v)
    vals_bufs = (vals0_v, vals1_v)
    out_bufs = (out0_v, out1_v)
    sems_g = (sem_g0, sem_g1)
    nch = out_hbm.shape[0] // _C
    tmax = (nch + _NW - 1) // _NW
    umax = (tmax + 1) // 2
    cid = lax.axis_index("c")
    sid = lax.axis_index("s")
    wid = sid * _NC + cid
    ins = (ox_hbm, oy_hbm, oz_hbm, vx_hbm, vy_hbm, vz_hbm)
    sems_in = (sem_in0, sem_in1)
    sems_out = (sem_out0, sem_out1)

    def fire_in(p, ch):
        base = ch * _C
        for comp in range(6):
            pltpu.async_copy(
                ins[comp].at[pl.ds(base, _C)], in_bufs[p][comp], sems_in[p])

    def wait_in(p, ch):
        base = ch * _C
        for comp in range(6):
            pltpu.make_async_copy(
                ins[comp].at[pl.ds(base, _C)], in_bufs[p][comp],
                sems_in[p]).wait()

    def compute_and_fire(p, ch):
        wait_in(p, ch)
        for r in range(_NGR):
            for s in range(_GPR):
                g = r * _GPR + s
                sl = pl.ds(g * 16, 16)
                ox = in_bufs[p][0][sl]
                oy = in_bufs[p][1][sl]
                oz = in_bufs[p][2][sl]
                vx = in_bufs[p][3][sl]
                vy = in_bufs[p][4][sl]
                vz = in_bufs[p][5][sl]
                m = jnp.maximum(jnp.maximum(jnp.abs(vx), jnp.abs(vy)),
                                jnp.abs(vz))
                m = jnp.maximum(m, jnp.float32(1e-12))
                rcp = jnp.float32(1.0) / m
                a = vx * rcp
                b = vy * rcp
                c = vz * rcp
                f = jnp.zeros((16,), jnp.int32)
                f = jnp.where(a <= -1.0, 1, f)
                f = jnp.where(b >= 1.0, 2, f)
                f = jnp.where(b <= -1.0, 3, f)
                f = jnp.where(c >= 1.0, 4, f)
                f = jnp.where(c <= -1.0, 5, f)
                ci = jnp.clip((ox * 0.5 + 0.5) * 127.0,
                              0.0, 127.0).astype(jnp.int32)
                cj = jnp.clip((oy * 0.5 + 0.5) * 127.0,
                              0.0, 127.0).astype(jnp.int32)
                ck = jnp.clip((oz * 0.5 + 0.5) * 127.0,
                              0.0, 127.0).astype(jnp.int32)
                flat = ((ci * 6 + f) * _GRID + cj) * _GRID + ck
                idx_bufs[p][r, pl.ds(s * 16, 16)] = flat
            pltpu.async_copy(
                cache_hbm.at[idx_bufs[p].at[r]], vals_bufs[p].at[r],
                sems_g[p])

    def drain_compare_out(p, ch, need_out_wait):
        for r in range(_NGR):
            pltpu.make_async_copy(
                cache_hbm.at[idx_bufs[p].at[r]], vals_bufs[p].at[r],
                sems_g[p]).wait()

        @pl.when(need_out_wait)
        def _():
            prev_base = (ch - 2 * _NW) * _C
            pltpu.make_async_copy(
                out_bufs[p], out_hbm.at[pl.ds(prev_base, _C)],
                sems_out[p]).wait()

        for r in range(_NGR):
            for s in range(_GPR):
                g = r * _GPR + s
                v = vals_bufs[p][r, pl.ds(s * 16, 16)]
                out_bufs[p][pl.ds(g * 16, 16)] = jnp.where(
                    v > _MID, 1, 0).astype(jnp.int32)
        pltpu.async_copy(
            out_bufs[p], out_hbm.at[pl.ds(ch * _C, _C)], sems_out[p])

    def phase(p, u, ch):
        @pl.when(ch + _NW < nch)
        def _():
            fire_in(1 - p, ch + _NW)

        compute_and_fire(p, ch)

        t_idx = 2 * u + p

        @pl.when(t_idx >= 1)
        def _():
            drain_compare_out(1 - p, ch - _NW, t_idx >= 3)

    @pl.when(wid < nch)
    def _():
        fire_in(0, wid)

    def iter_body(u, carry):
        ch_a = wid + (2 * u) * _NW
        ch_b = ch_a + _NW

        @pl.when(ch_a < nch)
        def _():
            phase(0, u, ch_a)

        @pl.when(ch_b < nch)
        def _():
            phase(1, u, ch_b)
        return carry

    lax.fori_loop(0, umax, iter_body, 0)

    q = (nch - 1 - wid) // _NW

    for qp in range(2):
        @pl.when((q >= 0) & ((q & 1) == qp))
        def _(qp=qp):
            drain_compare_out(qp, wid + q * _NW, q >= 2)

    for p in range(2):
        t_p = q - ((q - p) & 1)

        @pl.when(t_p >= 0)
        def _():
            last_base = (wid + t_p * _NW) * _C
            pltpu.make_async_copy(
                out_bufs[p], out_hbm.at[pl.ds(last_base, _C)],
                sems_out[p]).wait()


@jax.jit
def _vis_cache_sc(ox, oy, oz, vx, vy, vz, cache_flat):
    n = ox.shape[0]
    run = pl.kernel(
        _body,
        out_type=jax.ShapeDtypeStruct((n,), jnp.int32),
        mesh=plsc.VectorSubcoreMesh(core_axis_name="c", subcore_axis_name="s"),
        compiler_params=pltpu.CompilerParams(needs_layout_passes=False),
        scratch_types=[
            *([pltpu.VMEM((_C,), jnp.float32)] * 12),
            pltpu.VMEM((_NGR, _G), jnp.int32),
            pltpu.VMEM((_NGR, _G), jnp.int32),
            pltpu.VMEM((_NGR, _G), jnp.float32),
            pltpu.VMEM((_NGR, _G), jnp.float32),
            pltpu.VMEM((_C,), jnp.int32),
            pltpu.VMEM((_C,), jnp.int32),
            *([pltpu.SemaphoreType.DMA] * 6),
        ],
    )
    return run(ox, oy, oz, vx, vy, vz, cache_flat)


def kernel(norm_ray_origins, viewdirs, cache):
    n = norm_ray_origins.shape[0]
    denom = sum(_SEG_FRACS)
    segs = []
    acc = 0
    for i, fr in enumerate(_SEG_FRACS[:-1]):
        sz = (n * fr // denom // _C) * _C
        segs.append((acc, sz))
        acc += sz
    segs.append((acc, n - acc))
    assert all(sz % _C == 0 for _, sz in segs)
    cache_flat = cache.transpose(0, 3, 1, 2).reshape(-1)
    res = jnp.zeros((n,), jnp.bool_)
    fence = None
    o_src, v_src = norm_ray_origins, viewdirs
    for start, sz in segs:
        if fence is not None:
            o_src, v_src, *_ = lax.optimization_barrier(
                (norm_ray_origins, viewdirs, *fence))
        sl = slice(start, start + sz)
        cols = (
            o_src[sl, 0],
            o_src[sl, 1],
            o_src[sl, 2],
            v_src[sl, 0],
            v_src[sl, 1],
            v_src[sl, 2],
        )
        fence = cols
        seg_out = _vis_cache_sc(*cols, cache_flat) != 0
        res = lax.dynamic_update_slice(res, seg_out, (start,))
    return res


# speedup vs baseline: 1.2247x; 1.2247x over previous
import functools

import jax
import jax.numpy as jnp
from jax import lax
from jax.experimental import pallas as pl
from jax.experimental.pallas import tpu as pltpu
from jax.experimental.pallas import tpu_sc as plsc

_GRID = 128
_MID = 128.0

_C = 400
_G = 80
_SEG_FRACS = (3, 2)
_NGR = _C // _G
_GPR = _G // 16
_NC = 2
_NS = 16
_NW = _NC * _NS


def _body(ox_hbm, oy_hbm, oz_hbm, vx_hbm, vy_hbm, vz_hbm, cache_hbm, out_hbm,
          i00, i01, i02, i03, i04, i05,
          i10, i11, i12, i13, i14, i15,
          idx0_v, idx1_v, vals0_v, vals1_v, out0_v, out1_v,
          sem_in0, sem_in1, sem_g0, sem_g1, sem_out0, sem_out1):
    in_bufs = ((i00, i01, i02, i03, i04, i05),
               (i10, i11, i12, i13, i14, i15))
    idx_bufs = (idx0_v, idx1_v)
    vals_bufs = (vals0_v, vals1_v)
    out_bufs = (out0_v, out1_v)
    sems_g = (sem_g0, sem_g1)
    nch = out_hbm.shape[0] // _C
    tmax = (nch + _NW - 1) // _NW
    umax = (tmax + 1) // 2
    cid = lax.axis_index("c")
    sid = lax.axis_index("s")
    wid = sid * _NC + cid
    ins = (ox_hbm, oy_hbm, oz_hbm, vx_hbm, vy_hbm, vz_hbm)
    sems_in = (sem_in0, sem_in1)
    sems_out = (sem_out0, sem_out1)

    def fire_in(p, ch):
        base = ch * _C
        for comp in range(6):
            pltpu.async_copy(
                ins[comp].at[pl.ds(base, _C)], in_bufs[p][comp], sems_in[p])

    def wait_in(p, ch):
        base = ch * _C
        for comp in range(6):
            pltpu.make_async_copy(
                ins[comp].at[pl.ds(base, _C)], in_bufs[p][comp],
                sems_in[p]).wait()

    def compute_and_fire(p, ch):
        wait_in(p, ch)
        for r in range(_NGR):
            for s in range(_GPR):
                g = r * _GPR + s
                sl = pl.ds(g * 16, 16)
                ox = in_bufs[p][0][sl]
                oy = in_bufs[p][1][sl]
                oz = in_bufs[p][2][sl]
                vx = in_bufs[p][3][sl]
                vy = in_bufs[p][4][sl]
                vz = in_bufs[p][5][sl]
                m = jnp.maximum(jnp.maximum(jnp.abs(vx), jnp.abs(vy)),
                                jnp.abs(vz))
                m = jnp.maximum(m, jnp.float32(1e-12))
                rcp = jnp.float32(1.0) / m
                a = vx * rcp
                b = vy * rcp
                c = vz * rcp
                f = jnp.zeros((16,), jnp.int32)
                f = jnp.where(a <= -1.0, 1, f)
                f = jnp.where(b >= 1.0, 2, f)
                f = jnp.where(b <= -1.0, 3, f)
                f = jnp.where(c >= 1.0, 4, f)
                f = jnp.where(c <= -1.0, 5, f)
                ci = jnp.clip((ox * 0.5 + 0.5) * 127.0,
                              0.0, 127.0).astype(jnp.int32)
                cj = jnp.clip((oy * 0.5 + 0.5) * 127.0,
                              0.0, 127.0).astype(jnp.int32)
                ck = jnp.clip((oz * 0.5 + 0.5) * 127.0,
                              0.0, 127.0).astype(jnp.int32)
                flat = ((ci * 6 + f) * _GRID + cj) * _GRID + ck
                idx_bufs[p][r, pl.ds(s * 16, 16)] = flat
            pltpu.async_copy(
                cache_hbm.at[idx_bufs[p].at[r]], vals_bufs[p].at[r],
                sems_g[p])

    def drain_compare_out(p, ch, need_out_wait):
        for r in range(_NGR):
            pltpu.make_async_copy(
                cache_hbm.at[idx_bufs[p].at[r]], vals_bufs[p].at[r],
                sems_g[p]).wait()

        @pl.when(need_out_wait)
        def _():
            prev_base = (ch - 2 * _NW) * _C
            pltpu.make_async_copy(
                out_bufs[p], out_hbm.at[pl.ds(prev_base, _C)],
                sems_out[p]).wait()

        for r in range(_NGR):
            for s in range(_GPR):
                g = r * _GPR + s
                v = vals_bufs[p][r, pl.ds(s * 16, 16)]
                out_bufs[p][pl.ds(g * 16, 16)] = jnp.where(
                    v > _MID, 1, 0).astype(jnp.int32)
        pltpu.async_copy(
            out_bufs[p], out_hbm.at[pl.ds(ch * _C, _C)], sems_out[p])

    def phase(p, u, ch):
        @pl.when(ch + _NW < nch)
        def _():
            fire_in(1 - p, ch + _NW)

        compute_and_fire(p, ch)

        t_idx = 2 * u + p

        @pl.when(t_idx >= 1)
        def _():
            drain_compare_out(1 - p, ch - _NW, t_idx >= 3)

    @pl.when(wid < nch)
    def _():
        fire_in(0, wid)

    def iter_body(u, carry):
        ch_a = wid + (2 * u) * _NW
        ch_b = ch_a + _NW

        @pl.when(ch_a < nch)
        def _():
            phase(0, u, ch_a)

        @pl.when(ch_b < nch)
        def _():
            phase(1, u, ch_b)
        return carry

    lax.fori_loop(0, umax, iter_body, 0)

    q = (nch - 1 - wid) // _NW

    for qp in range(2):
        @pl.when((q >= 0) & ((q & 1) == qp))
        def _(qp=qp):
            drain_compare_out(qp, wid + q * _NW, q >= 2)

    for p in range(2):
        t_p = q - ((q - p) & 1)

        @pl.when(t_p >= 0)
        def _():
            last_base = (wid + t_p * _NW) * _C
            pltpu.make_async_copy(
                out_bufs[p], out_hbm.at[pl.ds(last_base, _C)],
                sems_out[p]).wait()


@jax.jit
def _vis_cache_sc(ox, oy, oz, vx, vy, vz, cache_flat):
    n = ox.shape[0]
    run = pl.kernel(
        _body,
        out_type=jax.ShapeDtypeStruct((n,), jnp.int32),
        mesh=plsc.VectorSubcoreMesh(core_axis_name="c", subcore_axis_name="s"),
        compiler_params=pltpu.CompilerParams(needs_layout_passes=False),
        scratch_types=[
            *([pltpu.VMEM((_C,), jnp.float32)] * 12),
            pltpu.VMEM((_NGR, _G), jnp.int32),
            pltpu.VMEM((_NGR, _G), jnp.int32),
            pltpu.VMEM((_NGR, _G), jnp.float32),
            pltpu.VMEM((_NGR, _G), jnp.float32),
            pltpu.VMEM((_C,), jnp.int32),
            pltpu.VMEM((_C,), jnp.int32),
            *([pltpu.SemaphoreType.DMA] * 6),
        ],
    )
    return run(ox, oy, oz, vx, vy, vz, cache_flat)


def kernel(norm_ray_origins, viewdirs, cache):
    n = norm_ray_origins.shape[0]
    denom = sum(_SEG_FRACS)
    segs = []
    acc = 0
    for i, fr in enumerate(_SEG_FRACS[:-1]):
        sz = (n * fr // denom // _C) * _C
        segs.append((acc, sz))
        acc += sz
    segs.append((acc, n - acc))
    assert all(sz % _C == 0 for _, sz in segs)
    cache_flat = cache.transpose(0, 3, 1, 2).reshape(-1)
    res = jnp.zeros((n,), jnp.bool_)
    fence = None
    o_src, v_src = norm_ray_origins, viewdirs
    for start, sz in segs:
        if fence is not None:
            o_src, v_src, *_ = lax.optimization_barrier(
                (norm_ray_origins, viewdirs, *fence))
        sl = slice(start, start + sz)
        cols = (
            o_src[sl, 0],
            o_src[sl, 1],
            o_src[sl, 2],
            v_src[sl, 0],
            v_src[sl, 1],
            v_src[sl, 2],
        )
        fence = cols
        seg_out = _vis_cache_sc(*cols, cache_flat) != 0
        res = lax.dynamic_update_slice(res, seg_out, (start,))
    return res
